# baseline (speedup 1.0000x reference)
import functools

import jax
import jax.numpy as jnp
from jax.experimental import pallas as pl
from jax.experimental.pallas import tpu as pltpu

B, N, C = 8, 4096, 32
G, K = 512, 32
D = 2 * C
GK = G * K
CHUNK = 4096
NCHUNK = GK // CHUNK


def _fps_jax(xyz):
    def body(i, carry):
        centroids, distance, farthest = carry
        centroids = centroids.at[:, i].set(farthest)
        centroid = jnp.take_along_axis(xyz, farthest[:, None, None], axis=1)
        dist = jnp.sum((xyz - centroid) ** 2, axis=-1)
        distance = jnp.minimum(distance, dist)
        farthest = jnp.argmax(distance, axis=-1).astype(jnp.int32)
        return (centroids, distance, farthest)

    centroids = jnp.zeros((B, G), dtype=jnp.int32)
    distance = jnp.full((B, N), 1e10, dtype=xyz.dtype)
    farthest = jnp.zeros((B,), dtype=jnp.int32)
    centroids, _, _ = jax.lax.fori_loop(0, G, body, (centroids, distance, farthest))
    return centroids


def _stats_kernel(g_ref, m_ref, s_ref, ss_ref):
    b = pl.program_id(0)
    c = pl.program_id(1)
    d = g_ref[0] - m_ref[0]

    @pl.when(c == 0)
    def _():
        s_ref[b] = 0.0
        ss_ref[b] = 0.0

    s_ref[b] += jnp.sum(d)
    ss_ref[b] += jnp.sum(d * d)


def _layer1_kernel(g_ref, m_ref, inv_ref, w_ref, y1_ref, s1_ref, ss1_ref):
    b = pl.program_id(0)
    c = pl.program_id(1)
    inv = inv_ref[b]
    d = (g_ref[0] - m_ref[0]) * inv
    x = jnp.concatenate([d, m_ref[0]], axis=1)
    y1 = jax.lax.dot_general(w_ref[...], x, (((1,), (1,)), ((), ())),
                             preferred_element_type=jnp.float32)
    y1_ref[0] = y1

    @pl.when(jnp.logical_and(b == 0, c == 0))
    def _():
        s1_ref[...] = jnp.zeros_like(s1_ref)
        ss1_ref[...] = jnp.zeros_like(ss1_ref)

    s1_ref[...] += jnp.sum(y1, axis=1)[None, :]
    ss1_ref[...] += jnp.sum(y1 * y1, axis=1)[None, :]


def _layer2_kernel(y1_ref, a1_ref, c1_ref, w_ref, y2_ref, s2_ref, ss2_ref):
    b = pl.program_id(0)
    c = pl.program_id(1)
    a1 = a1_ref[...]
    c1 = c1_ref[...]
    h1 = jnp.maximum(a1 * y1_ref[0] + c1, 0.0)
    y2 = jax.lax.dot_general(w_ref[...], h1, (((1,), (0,)), ((), ())),
                             preferred_element_type=jnp.float32)
    y2_ref[0] = y2

    @pl.when(jnp.logical_and(b == 0, c == 0))
    def _():
        s2_ref[...] = jnp.zeros_like(s2_ref)
        ss2_ref[...] = jnp.zeros_like(ss2_ref)

    s2_ref[...] += jnp.sum(y2, axis=1)[None, :]
    ss2_ref[...] += jnp.sum(y2 * y2, axis=1)[None, :]


def _finish_kernel(y2_ref, a2_ref, c2_ref, out_ref):
    a2 = a2_ref[...]
    c2 = c2_ref[...]
    h2 = jnp.maximum(a2 * y2_ref[0] + c2, 0.0)
    out_ref[0] = jnp.max(h2.reshape(D, CHUNK // K, K), axis=-1)


def _dense_stages(grouped2d, newp_rep, affine_alpha, affine_beta,
                  w1, g1, b1, w2, g2, b2):
    f32 = jnp.float32
    s, ss = pl.pallas_call(
        _stats_kernel,
        grid=(B, NCHUNK),
        in_specs=[
            pl.BlockSpec((1, CHUNK, C), lambda b, c: (b, c, 0)),
            pl.BlockSpec((1, CHUNK, C), lambda b, c: (b, c, 0)),
        ],
        out_specs=[
            pl.BlockSpec(memory_space=pltpu.SMEM),
            pl.BlockSpec(memory_space=pltpu.SMEM),
        ],
        out_shape=[jax.ShapeDtypeStruct((B,), f32),
                   jax.ShapeDtypeStruct((B,), f32)],
    )(grouped2d, newp_rep)
    n = float(G * K * C)
    std = jnp.sqrt((ss - s * s / n) / (n - 1.0))
    inv = 1.0 / (std + 1e-5)

    alpha = affine_alpha.reshape(C)
    w1a = w1[:, :C] * alpha[None, :]
    wcat = jnp.concatenate([w1a, w1[:, C:]], axis=1)

    y1, s1, ss1 = pl.pallas_call(
        _layer1_kernel,
        grid=(B, NCHUNK),
        in_specs=[
            pl.BlockSpec((1, CHUNK, C), lambda b, c: (b, c, 0)),
            pl.BlockSpec((1, CHUNK, C), lambda b, c: (b, c, 0)),
            pl.BlockSpec(memory_space=pltpu.SMEM),
            pl.BlockSpec((D, D), lambda b, c: (0, 0)),
        ],
        out_specs=[
            pl.BlockSpec((1, D, CHUNK), lambda b, c: (b, 0, c)),
            pl.BlockSpec((1, D), lambda b, c: (0, 0)),
            pl.BlockSpec((1, D), lambda b, c: (0, 0)),
        ],
        out_shape=[jax.ShapeDtypeStruct((B, D, GK), f32),
                   jax.ShapeDtypeStruct((1, D), f32),
                   jax.ShapeDtypeStruct((1, D), f32)],
    )(grouped2d, newp_rep, inv, wcat)

    n1 = float(B * GK)
    mu1 = s1[0] / n1
    var1 = ss1[0] / n1 - mu1 * mu1
    rs1 = 1.0 / jnp.sqrt(var1 + 1e-5)
    gamma1 = g1.reshape(D)
    a1 = (gamma1 * rs1).reshape(D, 1)
    c1 = (b1.reshape(D) - gamma1 * rs1 * mu1).reshape(D, 1)

    y2, s2, ss2 = pl.pallas_call(
        _layer2_kernel,
        grid=(B, NCHUNK),
        in_specs=[
            pl.BlockSpec((1, D, CHUNK), lambda b, c: (b, 0, c)),
            pl.BlockSpec((D, 1), lambda b, c: (0, 0)),
            pl.BlockSpec((D, 1), lambda b, c: (0, 0)),
            pl.BlockSpec((D, D), lambda b, c: (0, 0)),
        ],
        out_specs=[
            pl.BlockSpec((1, D, CHUNK), lambda b, c: (b, 0, c)),
            pl.BlockSpec((1, D), lambda b, c: (0, 0)),
            pl.BlockSpec((1, D), lambda b, c: (0, 0)),
        ],
        out_shape=[jax.ShapeDtypeStruct((B, D, GK), f32),
                   jax.ShapeDtypeStruct((1, D), f32),
                   jax.ShapeDtypeStruct((1, D), f32)],
    )(y1, a1, c1, w2)

    mu2 = s2[0] / n1
    var2 = ss2[0] / n1 - mu2 * mu2
    rs2 = 1.0 / jnp.sqrt(var2 + 1e-5)
    gamma2 = g2.reshape(D)
    a2 = (gamma2 * rs2).reshape(D, 1)
    c2 = (b2.reshape(D) - gamma2 * rs2 * mu2).reshape(D, 1)

    out = pl.pallas_call(
        _finish_kernel,
        grid=(B, NCHUNK),
        in_specs=[
            pl.BlockSpec((1, D, CHUNK), lambda b, c: (b, 0, c)),
            pl.BlockSpec((D, 1), lambda b, c: (0, 0)),
            pl.BlockSpec((D, 1), lambda b, c: (0, 0)),
        ],
        out_specs=pl.BlockSpec((1, D, CHUNK // K), lambda b, c: (b, 0, c)),
        out_shape=jax.ShapeDtypeStruct((B, D, G), f32),
    )(y2, a2, c2)
    return out


def kernel(xyz, points, affine_alpha, affine_beta, w1, g1, b1, w2, g2, b2):
    fps_idx = _fps_jax(jax.lax.stop_gradient(xyz))
    new_xyz = jax.vmap(lambda p, i: p[i])(xyz, fps_idx)
    new_points = jax.vmap(lambda p, i: p[i])(points, fps_idx)
    sqrdists = (jnp.sum(new_xyz ** 2, axis=-1)[..., None]
                + jnp.sum(xyz ** 2, axis=-1)[..., None, :]
                - 2.0 * jnp.matmul(new_xyz, jnp.swapaxes(xyz, 1, 2)))
    _, idx = jax.lax.top_k(-sqrdists, K)
    grouped = jax.vmap(lambda p, i: p[i])(points, idx)
    grouped2d = grouped.reshape(B, GK, C)
    newp_rep = jnp.repeat(new_points, K, axis=1)
    out = _dense_stages(grouped2d, newp_rep, affine_alpha, affine_beta,
                        w1, g1, b1, w2, g2, b2)
    return (new_xyz, out)


# speedup vs baseline: 1.8478x; 1.8478x over previous
import functools

import jax
import jax.numpy as jnp
from jax.experimental import pallas as pl
from jax.experimental.pallas import tpu as pltpu

B, N, C = 8, 4096, 32
G, K = 512, 32
D = 2 * C
GK = G * K
CHUNK = 4096
NCHUNK = GK // CHUNK


def _fps_kernel(xyzT_ref, cent_ref, nxyzT_ref):
    xs = xyzT_ref[0]
    ys = xyzT_ref[1]
    zs = xyzT_ref[2]
    col = jax.lax.broadcasted_iota(jnp.int32, (B, N), 1)
    gcol = jax.lax.broadcasted_iota(jnp.int32, (B, G), 1)

    def body(i, carry):
        dist, far, cent, nx, ny, nz = carry
        mask = col == far
        cx = jnp.sum(jnp.where(mask, xs, 0.0), axis=1, keepdims=True)
        cy = jnp.sum(jnp.where(mask, ys, 0.0), axis=1, keepdims=True)
        cz = jnp.sum(jnp.where(mask, zs, 0.0), axis=1, keepdims=True)
        rec = gcol == i
        cent = jnp.where(rec, far, cent)
        nx = jnp.where(rec, cx, nx)
        ny = jnp.where(rec, cy, ny)
        nz = jnp.where(rec, cz, nz)
        d = (xs - cx) ** 2 + (ys - cy) ** 2 + (zs - cz) ** 2
        dist = jnp.minimum(dist, d)
        m = jnp.max(dist, axis=1, keepdims=True)
        far = jnp.min(jnp.where(dist == m, col, N), axis=1, keepdims=True)
        return (dist, far, cent, nx, ny, nz)

    dist0 = xs * 0.0 + 1e10
    far0 = (xs[:, :1] * 0.0).astype(jnp.int32)
    nx0 = xs[:, :G] * 0.0
    cent0 = nx0.astype(jnp.int32)
    _, _, cent, nx, ny, nz = jax.lax.fori_loop(
        0, G, body, (dist0, far0, cent0, nx0, nx0 + 0.0, nx0 + 0.0))
    cent_ref[...] = cent
    nxyzT_ref[0] = nx
    nxyzT_ref[1] = ny
    nxyzT_ref[2] = nz


def _fps(xyzT):
    cent, nxyzT = pl.pallas_call(
        _fps_kernel,
        in_specs=[pl.BlockSpec((3, B, N), lambda: (0, 0, 0))],
        out_specs=[pl.BlockSpec((B, G), lambda: (0, 0)),
                   pl.BlockSpec((3, B, G), lambda: (0, 0, 0))],
        out_shape=[jax.ShapeDtypeStruct((B, G), jnp.int32),
                   jax.ShapeDtypeStruct((3, B, G), jnp.float32)],
    )(xyzT)
    return cent, nxyzT


def _stats_kernel(g_ref, m_ref, s_ref, ss_ref):
    b = pl.program_id(0)
    c = pl.program_id(1)
    d = g_ref[0] - m_ref[0]

    @pl.when(c == 0)
    def _():
        s_ref[b] = 0.0
        ss_ref[b] = 0.0

    s_ref[b] += jnp.sum(d)
    ss_ref[b] += jnp.sum(d * d)


def _layer1_kernel(g_ref, m_ref, inv_ref, w_ref, y1_ref, s1_ref, ss1_ref):
    b = pl.program_id(0)
    c = pl.program_id(1)
    inv = inv_ref[b]
    d = (g_ref[0] - m_ref[0]) * inv
    x = jnp.concatenate([d, m_ref[0]], axis=1)
    y1 = jax.lax.dot_general(w_ref[...], x, (((1,), (1,)), ((), ())),
                             preferred_element_type=jnp.float32)
    y1_ref[0] = y1

    @pl.when(jnp.logical_and(b == 0, c == 0))
    def _():
        s1_ref[...] = jnp.zeros_like(s1_ref)
        ss1_ref[...] = jnp.zeros_like(ss1_ref)

    s1_ref[...] += jnp.sum(y1, axis=1)[None, :]
    ss1_ref[...] += jnp.sum(y1 * y1, axis=1)[None, :]


def _layer2_kernel(y1_ref, a1_ref, c1_ref, w_ref, y2_ref, s2_ref, ss2_ref):
    b = pl.program_id(0)
    c = pl.program_id(1)
    a1 = a1_ref[...]
    c1 = c1_ref[...]
    h1 = jnp.maximum(a1 * y1_ref[0] + c1, 0.0)
    y2 = jax.lax.dot_general(w_ref[...], h1, (((1,), (0,)), ((), ())),
                             preferred_element_type=jnp.float32)
    y2_ref[0] = y2

    @pl.when(jnp.logical_and(b == 0, c == 0))
    def _():
        s2_ref[...] = jnp.zeros_like(s2_ref)
        ss2_ref[...] = jnp.zeros_like(ss2_ref)

    s2_ref[...] += jnp.sum(y2, axis=1)[None, :]
    ss2_ref[...] += jnp.sum(y2 * y2, axis=1)[None, :]


def _finish_kernel(y2_ref, a2_ref, c2_ref, out_ref):
    a2 = a2_ref[...]
    c2 = c2_ref[...]
    h2 = jnp.maximum(a2 * y2_ref[0] + c2, 0.0)
    out_ref[0] = jnp.max(h2.reshape(D, CHUNK // K, K), axis=-1)


def _dense_stages(grouped2d, newp_rep, affine_alpha, affine_beta,
                  w1, g1, b1, w2, g2, b2):
    f32 = jnp.float32
    s, ss = pl.pallas_call(
        _stats_kernel,
        grid=(B, NCHUNK),
        in_specs=[
            pl.BlockSpec((1, CHUNK, C), lambda b, c: (b, c, 0)),
            pl.BlockSpec((1, CHUNK, C), lambda b, c: (b, c, 0)),
        ],
        out_specs=[
            pl.BlockSpec(memory_space=pltpu.SMEM),
            pl.BlockSpec(memory_space=pltpu.SMEM),
        ],
        out_shape=[jax.ShapeDtypeStruct((B,), f32),
                   jax.ShapeDtypeStruct((B,), f32)],
    )(grouped2d, newp_rep)
    n = float(G * K * C)
    std = jnp.sqrt((ss - s * s / n) / (n - 1.0))
    inv = 1.0 / (std + 1e-5)

    alpha = affine_alpha.reshape(C)
    w1a = w1[:, :C] * alpha[None, :]
    wcat = jnp.concatenate([w1a, w1[:, C:]], axis=1)

    y1, s1, ss1 = pl.pallas_call(
        _layer1_kernel,
        grid=(B, NCHUNK),
        in_specs=[
            pl.BlockSpec((1, CHUNK, C), lambda b, c: (b, c, 0)),
            pl.BlockSpec((1, CHUNK, C), lambda b, c: (b, c, 0)),
            pl.BlockSpec(memory_space=pltpu.SMEM),
            pl.BlockSpec((D, D), lambda b, c: (0, 0)),
        ],
        out_specs=[
            pl.BlockSpec((1, D, CHUNK), lambda b, c: (b, 0, c)),
            pl.BlockSpec((1, D), lambda b, c: (0, 0)),
            pl.BlockSpec((1, D), lambda b, c: (0, 0)),
        ],
        out_shape=[jax.ShapeDtypeStruct((B, D, GK), f32),
                   jax.ShapeDtypeStruct((1, D), f32),
                   jax.ShapeDtypeStruct((1, D), f32)],
    )(grouped2d, newp_rep, inv, wcat)

    n1 = float(B * GK)
    mu1 = s1[0] / n1
    var1 = ss1[0] / n1 - mu1 * mu1
    rs1 = 1.0 / jnp.sqrt(var1 + 1e-5)
    gamma1 = g1.reshape(D)
    a1 = (gamma1 * rs1).reshape(D, 1)
    c1 = (b1.reshape(D) - gamma1 * rs1 * mu1).reshape(D, 1)

    y2, s2, ss2 = pl.pallas_call(
        _layer2_kernel,
        grid=(B, NCHUNK),
        in_specs=[
            pl.BlockSpec((1, D, CHUNK), lambda b, c: (b, 0, c)),
            pl.BlockSpec((D, 1), lambda b, c: (0, 0)),
            pl.BlockSpec((D, 1), lambda b, c: (0, 0)),
            pl.BlockSpec((D, D), lambda b, c: (0, 0)),
        ],
        out_specs=[
            pl.BlockSpec((1, D, CHUNK), lambda b, c: (b, 0, c)),
            pl.BlockSpec((1, D), lambda b, c: (0, 0)),
            pl.BlockSpec((1, D), lambda b, c: (0, 0)),
        ],
        out_shape=[jax.ShapeDtypeStruct((B, D, GK), f32),
                   jax.ShapeDtypeStruct((1, D), f32),
                   jax.ShapeDtypeStruct((1, D), f32)],
    )(y1, a1, c1, w2)

    mu2 = s2[0] / n1
    var2 = ss2[0] / n1 - mu2 * mu2
    rs2 = 1.0 / jnp.sqrt(var2 + 1e-5)
    gamma2 = g2.reshape(D)
    a2 = (gamma2 * rs2).reshape(D, 1)
    c2 = (b2.reshape(D) - gamma2 * rs2 * mu2).reshape(D, 1)

    out = pl.pallas_call(
        _finish_kernel,
        grid=(B, NCHUNK),
        in_specs=[
            pl.BlockSpec((1, D, CHUNK), lambda b, c: (b, 0, c)),
            pl.BlockSpec((D, 1), lambda b, c: (0, 0)),
            pl.BlockSpec((D, 1), lambda b, c: (0, 0)),
        ],
        out_specs=pl.BlockSpec((1, D, CHUNK // K), lambda b, c: (b, 0, c)),
        out_shape=jax.ShapeDtypeStruct((B, D, G), f32),
    )(y2, a2, c2)
    return out


def kernel(xyz, points, affine_alpha, affine_beta, w1, g1, b1, w2, g2, b2):
    xyzT = jnp.transpose(xyz, (2, 0, 1))
    fps_idx, nxyzT = _fps(xyzT)
    new_xyz = jnp.transpose(nxyzT, (1, 2, 0))
    new_points = jax.vmap(lambda p, i: p[i])(points, fps_idx)
    sqrdists = (jnp.sum(new_xyz ** 2, axis=-1)[..., None]
                + jnp.sum(xyz ** 2, axis=-1)[..., None, :]
                - 2.0 * jnp.matmul(new_xyz, jnp.swapaxes(xyz, 1, 2)))
    _, idx = jax.lax.top_k(-sqrdists, K)
    grouped = jax.vmap(lambda p, i: p[i])(points, idx)
    grouped2d = grouped.reshape(B, GK, C)
    newp_rep = jnp.repeat(new_points, K, axis=1)
    out = _dense_stages(grouped2d, newp_rep, affine_alpha, affine_beta,
                        w1, g1, b1, w2, g2, b2)
    return (new_xyz, out)


# speedup vs baseline: 2.9836x; 1.6147x over previous
import functools

import jax
import jax.numpy as jnp
from jax.experimental import pallas as pl
from jax.experimental.pallas import tpu as pltpu

B, N, C = 8, 4096, 32
G, K = 512, 32
D = 2 * C
GK = G * K
CHUNK = 4096
NCHUNK = GK // CHUNK


def _fps_kernel(xyzT_ref, cent_ref, nxyzT_ref):
    xs = xyzT_ref[0]
    ys = xyzT_ref[1]
    zs = xyzT_ref[2]
    col = jax.lax.broadcasted_iota(jnp.int32, (B, N), 1)
    gcol = jax.lax.broadcasted_iota(jnp.int32, (B, G), 1)

    def body(i, carry):
        dist, far, cent, nx, ny, nz = carry
        mask = col == far
        cx = jnp.sum(jnp.where(mask, xs, 0.0), axis=1, keepdims=True)
        cy = jnp.sum(jnp.where(mask, ys, 0.0), axis=1, keepdims=True)
        cz = jnp.sum(jnp.where(mask, zs, 0.0), axis=1, keepdims=True)
        rec = gcol == i
        cent = jnp.where(rec, far, cent)
        nx = jnp.where(rec, cx, nx)
        ny = jnp.where(rec, cy, ny)
        nz = jnp.where(rec, cz, nz)
        d = (xs - cx) ** 2 + (ys - cy) ** 2 + (zs - cz) ** 2
        dist = jnp.minimum(dist, d)
        m = jnp.max(dist, axis=1, keepdims=True)
        far = jnp.min(jnp.where(dist == m, col, N), axis=1, keepdims=True)
        return (dist, far, cent, nx, ny, nz)

    dist0 = xs * 0.0 + 1e10
    far0 = (xs[:, :1] * 0.0).astype(jnp.int32)
    nx0 = xs[:, :G] * 0.0
    cent0 = nx0.astype(jnp.int32)
    _, _, cent, nx, ny, nz = jax.lax.fori_loop(
        0, G, body, (dist0, far0, cent0, nx0, nx0 + 0.0, nx0 + 0.0))
    cent_ref[...] = cent
    nxyzT_ref[0] = nx
    nxyzT_ref[1] = ny
    nxyzT_ref[2] = nz


def _fps(xyzT):
    cent, nxyzT = pl.pallas_call(
        _fps_kernel,
        in_specs=[pl.BlockSpec((3, B, N), lambda: (0, 0, 0))],
        out_specs=[pl.BlockSpec((B, G), lambda: (0, 0)),
                   pl.BlockSpec((3, B, G), lambda: (0, 0, 0))],
        out_shape=[jax.ShapeDtypeStruct((B, G), jnp.int32),
                   jax.ShapeDtypeStruct((3, B, G), jnp.float32)],
    )(xyzT)
    return cent, nxyzT


def _knn_kernel(xyzT_ref, nxyz_ref, idx_ref):
    t = xyzT_ref[0]
    xs = t[0:1]
    ys = t[1:2]
    zs = t[2:3]
    nxyz = nxyz_ref[0]
    sa = jnp.sum(nxyz * nxyz, axis=1, keepdims=True)
    sx = xs * xs + ys * ys + zs * zs
    def _dotf(a, b):
        return jax.lax.dot_general(a, b, (((1,), (0,)), ((), ())),
                                   preferred_element_type=jnp.float32)

    mm = _dotf(nxyz, t)
    dist = (sa + sx) - 2.0 * mm
    col = jax.lax.broadcasted_iota(jnp.int32, (G, N), 1)
    kcol = jax.lax.broadcasted_iota(jnp.int32, (G, K), 1)

    def body(k, carry):
        dist, idxacc = carry
        m = jnp.min(dist, axis=1, keepdims=True)
        hit = dist == m
        amin = jnp.min(jnp.where(hit, col, N), axis=1, keepdims=True)
        idxacc = jnp.where(kcol == k, amin, idxacc)
        dist = jnp.where(jnp.logical_and(hit, col == amin), 1e30, dist)
        return (dist, idxacc)

    idx0 = (dist[:, :K] * 0.0).astype(jnp.int32)
    _, idxacc = jax.lax.fori_loop(0, K, body, (dist, idx0))
    idx_ref[0] = idxacc


def _knn(xyzT, new_xyz):
    return pl.pallas_call(
        _knn_kernel,
        grid=(B,),
        in_specs=[pl.BlockSpec((1, 3, N), lambda b: (b, 0, 0)),
                  pl.BlockSpec((1, G, 3), lambda b: (b, 0, 0))],
        out_specs=pl.BlockSpec((1, G, K), lambda b: (b, 0, 0)),
        out_shape=jax.ShapeDtypeStruct((B, G, K), jnp.int32),
    )(xyzT, new_xyz)


def _stats_kernel(g_ref, m_ref, s_ref, ss_ref):
    b = pl.program_id(0)
    c = pl.program_id(1)
    d = g_ref[0] - m_ref[0]

    @pl.when(c == 0)
    def _():
        s_ref[b] = 0.0
        ss_ref[b] = 0.0

    s_ref[b] += jnp.sum(d)
    ss_ref[b] += jnp.sum(d * d)


def _layer1_kernel(g_ref, m_ref, inv_ref, w_ref, y1_ref, s1_ref, ss1_ref):
    b = pl.program_id(0)
    c = pl.program_id(1)
    inv = inv_ref[b]
    d = (g_ref[0] - m_ref[0]) * inv
    x = jnp.concatenate([d, m_ref[0]], axis=1)
    y1 = jax.lax.dot_general(w_ref[...], x, (((1,), (1,)), ((), ())),
                             preferred_element_type=jnp.float32)
    y1_ref[0] = y1

    @pl.when(jnp.logical_and(b == 0, c == 0))
    def _():
        s1_ref[...] = jnp.zeros_like(s1_ref)
        ss1_ref[...] = jnp.zeros_like(ss1_ref)

    s1_ref[...] += jnp.sum(y1, axis=1)[None, :]
    ss1_ref[...] += jnp.sum(y1 * y1, axis=1)[None, :]


def _layer2_kernel(y1_ref, a1_ref, c1_ref, w_ref, y2_ref, s2_ref, ss2_ref):
    b = pl.program_id(0)
    c = pl.program_id(1)
    a1 = a1_ref[...]
    c1 = c1_ref[...]
    h1 = jnp.maximum(a1 * y1_ref[0] + c1, 0.0)
    y2 = jax.lax.dot_general(w_ref[...], h1, (((1,), (0,)), ((), ())),
                             preferred_element_type=jnp.float32)
    y2_ref[0] = y2

    @pl.when(jnp.logical_and(b == 0, c == 0))
    def _():
        s2_ref[...] = jnp.zeros_like(s2_ref)
        ss2_ref[...] = jnp.zeros_like(ss2_ref)

    s2_ref[...] += jnp.sum(y2, axis=1)[None, :]
    ss2_ref[...] += jnp.sum(y2 * y2, axis=1)[None, :]


def _finish_kernel(y2_ref, a2_ref, c2_ref, out_ref):
    a2 = a2_ref[...]
    c2 = c2_ref[...]
    h2 = jnp.maximum(a2 * y2_ref[0] + c2, 0.0)
    out_ref[0] = jnp.max(h2.reshape(D, CHUNK // K, K), axis=-1)


def _dense_stages(grouped2d, newp_rep, affine_alpha, affine_beta,
                  w1, g1, b1, w2, g2, b2):
    f32 = jnp.float32
    s, ss = pl.pallas_call(
        _stats_kernel,
        grid=(B, NCHUNK),
        in_specs=[
            pl.BlockSpec((1, CHUNK, C), lambda b, c: (b, c, 0)),
            pl.BlockSpec((1, CHUNK, C), lambda b, c: (b, c, 0)),
        ],
        out_specs=[
            pl.BlockSpec(memory_space=pltpu.SMEM),
            pl.BlockSpec(memory_space=pltpu.SMEM),
        ],
        out_shape=[jax.ShapeDtypeStruct((B,), f32),
                   jax.ShapeDtypeStruct((B,), f32)],
    )(grouped2d, newp_rep)
    n = float(G * K * C)
    std = jnp.sqrt((ss - s * s / n) / (n - 1.0))
    inv = 1.0 / (std + 1e-5)

    alpha = affine_alpha.reshape(C)
    w1a = w1[:, :C] * alpha[None, :]
    wcat = jnp.concatenate([w1a, w1[:, C:]], axis=1)

    y1, s1, ss1 = pl.pallas_call(
        _layer1_kernel,
        grid=(B, NCHUNK),
        in_specs=[
            pl.BlockSpec((1, CHUNK, C), lambda b, c: (b, c, 0)),
            pl.BlockSpec((1, CHUNK, C), lambda b, c: (b, c, 0)),
            pl.BlockSpec(memory_space=pltpu.SMEM),
            pl.BlockSpec((D, D), lambda b, c: (0, 0)),
        ],
        out_specs=[
            pl.BlockSpec((1, D, CHUNK), lambda b, c: (b, 0, c)),
            pl.BlockSpec((1, D), lambda b, c: (0, 0)),
            pl.BlockSpec((1, D), lambda b, c: (0, 0)),
        ],
        out_shape=[jax.ShapeDtypeStruct((B, D, GK), f32),
                   jax.ShapeDtypeStruct((1, D), f32),
                   jax.ShapeDtypeStruct((1, D), f32)],
    )(grouped2d, newp_rep, inv, wcat)

    n1 = float(B * GK)
    mu1 = s1[0] / n1
    var1 = ss1[0] / n1 - mu1 * mu1
    rs1 = 1.0 / jnp.sqrt(var1 + 1e-5)
    gamma1 = g1.reshape(D)
    a1 = (gamma1 * rs1).reshape(D, 1)
    c1 = (b1.reshape(D) - gamma1 * rs1 * mu1).reshape(D, 1)

    y2, s2, ss2 = pl.pallas_call(
        _layer2_kernel,
        grid=(B, NCHUNK),
        in_specs=[
            pl.BlockSpec((1, D, CHUNK), lambda b, c: (b, 0, c)),
            pl.BlockSpec((D, 1), lambda b, c: (0, 0)),
            pl.BlockSpec((D, 1), lambda b, c: (0, 0)),
            pl.BlockSpec((D, D), lambda b, c: (0, 0)),
        ],
        out_specs=[
            pl.BlockSpec((1, D, CHUNK), lambda b, c: (b, 0, c)),
            pl.BlockSpec((1, D), lambda b, c: (0, 0)),
            pl.BlockSpec((1, D), lambda b, c: (0, 0)),
        ],
        out_shape=[jax.ShapeDtypeStruct((B, D, GK), f32),
                   jax.ShapeDtypeStruct((1, D), f32),
                   jax.ShapeDtypeStruct((1, D), f32)],
    )(y1, a1, c1, w2)

    mu2 = s2[0] / n1
    var2 = ss2[0] / n1 - mu2 * mu2
    rs2 = 1.0 / jnp.sqrt(var2 + 1e-5)
    gamma2 = g2.reshape(D)
    a2 = (gamma2 * rs2).reshape(D, 1)
    c2 = (b2.reshape(D) - gamma2 * rs2 * mu2).reshape(D, 1)

    out = pl.pallas_call(
        _finish_kernel,
        grid=(B, NCHUNK),
        in_specs=[
            pl.BlockSpec((1, D, CHUNK), lambda b, c: (b, 0, c)),
            pl.BlockSpec((D, 1), lambda b, c: (0, 0)),
            pl.BlockSpec((D, 1), lambda b, c: (0, 0)),
        ],
        out_specs=pl.BlockSpec((1, D, CHUNK // K), lambda b, c: (b, 0, c)),
        out_shape=jax.ShapeDtypeStruct((B, D, G), f32),
    )(y2, a2, c2)
    return out


def kernel(xyz, points, affine_alpha, affine_beta, w1, g1, b1, w2, g2, b2):
    xyzT = jnp.transpose(xyz, (2, 0, 1))
    fps_idx, nxyzT = _fps(xyzT)
    new_xyz = jnp.transpose(nxyzT, (1, 2, 0))
    new_points = jax.vmap(lambda p, i: p[i])(points, fps_idx)
    idx = _knn(jnp.transpose(xyz, (0, 2, 1)), new_xyz)
    grouped = jax.vmap(lambda p, i: p[i])(points, idx)
    grouped2d = grouped.reshape(B, GK, C)
    newp_rep = jnp.repeat(new_points, K, axis=1)
    out = _dense_stages(grouped2d, newp_rep, affine_alpha, affine_beta,
                        w1, g1, b1, w2, g2, b2)
    return (new_xyz, out)


# speedup vs baseline: 6.7299x; 2.2556x over previous
import functools

import jax
import jax.numpy as jnp
from jax import lax
from jax.experimental import pallas as pl
from jax.experimental.pallas import tpu as pltpu
from jax.experimental.pallas import tpu_sc as plsc

B, N, C = 8, 4096, 32
G, K = 512, 32
D = 2 * C
GK = G * K
CHUNK = 4096
NCHUNK = GK // CHUNK


def _fps_kernel(xyzT_ref, cent_ref, nxyzT_ref):
    xs = xyzT_ref[0]
    ys = xyzT_ref[1]
    zs = xyzT_ref[2]
    col = jax.lax.broadcasted_iota(jnp.int32, (B, N), 1)
    gcol = jax.lax.broadcasted_iota(jnp.int32, (B, G), 1)

    def body(i, carry):
        dist, far, cent, nx, ny, nz = carry
        mask = col == far
        cx = jnp.sum(jnp.where(mask, xs, 0.0), axis=1, keepdims=True)
        cy = jnp.sum(jnp.where(mask, ys, 0.0), axis=1, keepdims=True)
        cz = jnp.sum(jnp.where(mask, zs, 0.0), axis=1, keepdims=True)
        rec = gcol == i
        cent = jnp.where(rec, far, cent)
        nx = jnp.where(rec, cx, nx)
        ny = jnp.where(rec, cy, ny)
        nz = jnp.where(rec, cz, nz)
        d = (xs - cx) ** 2 + (ys - cy) ** 2 + (zs - cz) ** 2
        dist = jnp.minimum(dist, d)
        m = jnp.max(dist, axis=1, keepdims=True)
        far = jnp.min(jnp.where(dist == m, col, N), axis=1, keepdims=True)
        return (dist, far, cent, nx, ny, nz)

    dist0 = xs * 0.0 + 1e10
    far0 = (xs[:, :1] * 0.0).astype(jnp.int32)
    nx0 = xs[:, :G] * 0.0
    cent0 = nx0.astype(jnp.int32)
    _, _, cent, nx, ny, nz = jax.lax.fori_loop(
        0, G, body, (dist0, far0, cent0, nx0, nx0 + 0.0, nx0 + 0.0))
    cent_ref[...] = cent + N * jax.lax.broadcasted_iota(jnp.int32, (B, G), 0)
    nxyzT_ref[0] = nx
    nxyzT_ref[1] = ny
    nxyzT_ref[2] = nz


def _fps(xyzT):
    cent, nxyzT = pl.pallas_call(
        _fps_kernel,
        in_specs=[pl.BlockSpec((3, B, N), lambda: (0, 0, 0))],
        out_specs=[pl.BlockSpec((B, G), lambda: (0, 0)),
                   pl.BlockSpec((3, B, G), lambda: (0, 0, 0))],
        out_shape=[jax.ShapeDtypeStruct((B, G), jnp.int32),
                   jax.ShapeDtypeStruct((3, B, G), jnp.float32)],
    )(xyzT)
    return cent, nxyzT


def _knn_kernel(xyzT_ref, nxyz_ref, idx_ref):
    t = xyzT_ref[0]
    xs = t[0:1]
    ys = t[1:2]
    zs = t[2:3]
    nxyz = nxyz_ref[0]
    sa = jnp.sum(nxyz * nxyz, axis=1, keepdims=True)
    sx = xs * xs + ys * ys + zs * zs
    def _dotf(a, b):
        return jax.lax.dot_general(a, b, (((1,), (0,)), ((), ())),
                                   preferred_element_type=jnp.float32)

    mm = _dotf(nxyz, t)
    dist = (sa + sx) - 2.0 * mm
    col = jax.lax.broadcasted_iota(jnp.int32, (G, N), 1)
    kcol = jax.lax.broadcasted_iota(jnp.int32, (G, K), 1)

    def body(k, carry):
        dist, idxacc = carry
        m = jnp.min(dist, axis=1, keepdims=True)
        hit = dist == m
        amin = jnp.min(jnp.where(hit, col, N), axis=1, keepdims=True)
        idxacc = jnp.where(kcol == k, amin, idxacc)
        dist = jnp.where(jnp.logical_and(hit, col == amin), 1e30, dist)
        return (dist, idxacc)

    idx0 = (dist[:, :K] * 0.0).astype(jnp.int32)
    _, idxacc = jax.lax.fori_loop(0, K, body, (dist, idx0))
    idx_ref[0] = idxacc + N * pl.program_id(0)


def _knn(xyzT, new_xyz):
    return pl.pallas_call(
        _knn_kernel,
        grid=(B,),
        in_specs=[pl.BlockSpec((1, 3, N), lambda b: (b, 0, 0)),
                  pl.BlockSpec((1, G, 3), lambda b: (b, 0, 0))],
        out_specs=pl.BlockSpec((1, G, K), lambda b: (b, 0, 0)),
        out_shape=jax.ShapeDtypeStruct((B, G, K), jnp.int32),
    )(xyzT, new_xyz)


def _sc_gather(points_flat, gidx_flat, n_rows, chunk):
    info = plsc.get_sparse_core_info()
    nw = info.num_cores * info.num_subcores
    rows_per_w = n_rows // nw
    n_chunks = rows_per_w // chunk
    mesh = plsc.VectorSubcoreMesh(core_axis_name="c", subcore_axis_name="s")

    @functools.partial(
        pl.kernel, mesh=mesh,
        out_type=jax.ShapeDtypeStruct((n_rows, C), jnp.float32),
        compiler_params=pltpu.CompilerParams(use_tc_tiling_on_sc=False),
        scratch_types=[
            pltpu.VMEM((chunk,), jnp.int32),
            pltpu.VMEM((chunk, C), jnp.float32),
            pltpu.SemaphoreType.DMA,
        ],
    )
    def k(points_hbm, idx_hbm, out_hbm, idx_v, rows_v, sem):
        wid = lax.axis_index("s") * info.num_cores + lax.axis_index("c")
        base = wid * rows_per_w

        def chunk_body(j, carry):
            off = base + j * chunk
            pltpu.sync_copy(idx_hbm.at[pl.ds(off, chunk)], idx_v)
            pltpu.async_copy(points_hbm.at[idx_v], rows_v, sem).wait()
            pltpu.sync_copy(rows_v, out_hbm.at[pl.ds(off, chunk)])
            return carry

        lax.fori_loop(0, n_chunks, chunk_body, 0)

    return k(points_flat, gidx_flat)


GB = CHUNK // K


def _expand_mean(npc):
    return jnp.broadcast_to(npc[:, None, :], (GB, K, C)).reshape(CHUNK, C)


def _stats_kernel(g_ref, m_ref, s_ref, ss_ref):
    b = pl.program_id(0)
    c = pl.program_id(1)
    d = g_ref[0] - _expand_mean(m_ref[0])

    @pl.when(c == 0)
    def _():
        s_ref[b] = 0.0
        ss_ref[b] = 0.0

    s_ref[b] += jnp.sum(d)
    ss_ref[b] += jnp.sum(d * d)


def _layer1_kernel(g_ref, m_ref, inv_ref, w_ref, y1_ref, s1_ref, ss1_ref):
    b = pl.program_id(0)
    c = pl.program_id(1)
    inv = inv_ref[b]
    m = _expand_mean(m_ref[0])
    d = (g_ref[0] - m) * inv
    x = jnp.concatenate([d, m], axis=1)
    y1 = jax.lax.dot_general(w_ref[...], x, (((1,), (1,)), ((), ())),
                             preferred_element_type=jnp.float32)
    y1_ref[0] = y1

    @pl.when(jnp.logical_and(b == 0, c == 0))
    def _():
        s1_ref[...] = jnp.zeros_like(s1_ref)
        ss1_ref[...] = jnp.zeros_like(ss1_ref)

    s1_ref[...] += jnp.sum(y1, axis=1)[None, :]
    ss1_ref[...] += jnp.sum(y1 * y1, axis=1)[None, :]


def _layer2_kernel(y1_ref, a1_ref, c1_ref, w_ref, y2_ref, s2_ref, ss2_ref):
    b = pl.program_id(0)
    c = pl.program_id(1)
    a1 = a1_ref[...]
    c1 = c1_ref[...]
    h1 = jnp.maximum(a1 * y1_ref[0] + c1, 0.0)
    y2 = jax.lax.dot_general(w_ref[...], h1, (((1,), (0,)), ((), ())),
                             preferred_element_type=jnp.float32)
    y2_ref[0] = y2

    @pl.when(jnp.logical_and(b == 0, c == 0))
    def _():
        s2_ref[...] = jnp.zeros_like(s2_ref)
        ss2_ref[...] = jnp.zeros_like(ss2_ref)

    s2_ref[...] += jnp.sum(y2, axis=1)[None, :]
    ss2_ref[...] += jnp.sum(y2 * y2, axis=1)[None, :]


def _finish_kernel(y2_ref, a2_ref, c2_ref, out_ref):
    a2 = a2_ref[...]
    c2 = c2_ref[...]
    h2 = jnp.maximum(a2 * y2_ref[0] + c2, 0.0)
    out_ref[0] = jnp.max(h2.reshape(D, CHUNK // K, K), axis=-1)


def _dense_stages(grouped2d, new_points, affine_alpha, affine_beta,
                  w1, g1, b1, w2, g2, b2):
    f32 = jnp.float32
    s, ss = pl.pallas_call(
        _stats_kernel,
        grid=(B, NCHUNK),
        in_specs=[
            pl.BlockSpec((1, CHUNK, C), lambda b, c: (b, c, 0)),
            pl.BlockSpec((1, GB, C), lambda b, c: (b, c, 0)),
        ],
        out_specs=[
            pl.BlockSpec(memory_space=pltpu.SMEM),
            pl.BlockSpec(memory_space=pltpu.SMEM),
        ],
        out_shape=[jax.ShapeDtypeStruct((B,), f32),
                   jax.ShapeDtypeStruct((B,), f32)],
    )(grouped2d, new_points)
    n = float(G * K * C)
    std = jnp.sqrt((ss - s * s / n) / (n - 1.0))
    inv = 1.0 / (std + 1e-5)

    alpha = affine_alpha.reshape(C)
    w1a = w1[:, :C] * alpha[None, :]
    wcat = jnp.concatenate([w1a, w1[:, C:]], axis=1)

    y1, s1, ss1 = pl.pallas_call(
        _layer1_kernel,
        grid=(B, NCHUNK),
        in_specs=[
            pl.BlockSpec((1, CHUNK, C), lambda b, c: (b, c, 0)),
            pl.BlockSpec((1, GB, C), lambda b, c: (b, c, 0)),
            pl.BlockSpec(memory_space=pltpu.SMEM),
            pl.BlockSpec((D, D), lambda b, c: (0, 0)),
        ],
        out_specs=[
            pl.BlockSpec((1, D, CHUNK), lambda b, c: (b, 0, c)),
            pl.BlockSpec((1, D), lambda b, c: (0, 0)),
            pl.BlockSpec((1, D), lambda b, c: (0, 0)),
        ],
        out_shape=[jax.ShapeDtypeStruct((B, D, GK), f32),
                   jax.ShapeDtypeStruct((1, D), f32),
                   jax.ShapeDtypeStruct((1, D), f32)],
    )(grouped2d, new_points, inv, wcat)

    n1 = float(B * GK)
    mu1 = s1[0] / n1
    var1 = ss1[0] / n1 - mu1 * mu1
    rs1 = 1.0 / jnp.sqrt(var1 + 1e-5)
    gamma1 = g1.reshape(D)
    a1 = (gamma1 * rs1).reshape(D, 1)
    c1 = (b1.reshape(D) - gamma1 * rs1 * mu1).reshape(D, 1)

    y2, s2, ss2 = pl.pallas_call(
        _layer2_kernel,
        grid=(B, NCHUNK),
        in_specs=[
            pl.BlockSpec((1, D, CHUNK), lambda b, c: (b, 0, c)),
            pl.BlockSpec((D, 1), lambda b, c: (0, 0)),
            pl.BlockSpec((D, 1), lambda b, c: (0, 0)),
            pl.BlockSpec((D, D), lambda b, c: (0, 0)),
        ],
        out_specs=[
            pl.BlockSpec((1, D, CHUNK), lambda b, c: (b, 0, c)),
            pl.BlockSpec((1, D), lambda b, c: (0, 0)),
            pl.BlockSpec((1, D), lambda b, c: (0, 0)),
        ],
        out_shape=[jax.ShapeDtypeStruct((B, D, GK), f32),
                   jax.ShapeDtypeStruct((1, D), f32),
                   jax.ShapeDtypeStruct((1, D), f32)],
    )(y1, a1, c1, w2)

    mu2 = s2[0] / n1
    var2 = ss2[0] / n1 - mu2 * mu2
    rs2 = 1.0 / jnp.sqrt(var2 + 1e-5)
    gamma2 = g2.reshape(D)
    a2 = (gamma2 * rs2).reshape(D, 1)
    c2 = (b2.reshape(D) - gamma2 * rs2 * mu2).reshape(D, 1)

    out = pl.pallas_call(
        _finish_kernel,
        grid=(B, NCHUNK),
        in_specs=[
            pl.BlockSpec((1, D, CHUNK), lambda b, c: (b, 0, c)),
            pl.BlockSpec((D, 1), lambda b, c: (0, 0)),
            pl.BlockSpec((D, 1), lambda b, c: (0, 0)),
        ],
        out_specs=pl.BlockSpec((1, D, CHUNK // K), lambda b, c: (b, 0, c)),
        out_shape=jax.ShapeDtypeStruct((B, D, G), f32),
    )(y2, a2, c2)
    return out


def kernel(xyz, points, affine_alpha, affine_beta, w1, g1, b1, w2, g2, b2):
    xyzT = jnp.transpose(xyz, (2, 0, 1))
    fps_idx, nxyzT = _fps(xyzT)
    new_xyz = jnp.transpose(nxyzT, (1, 2, 0))
    idx = _knn(jnp.transpose(xyz, (0, 2, 1)), new_xyz)
    points_flat = points.reshape(B * N, C)
    new_points = _sc_gather(points_flat, fps_idx.reshape(B * G),
                            B * G, 128).reshape(B, G, C)
    grouped2d = _sc_gather(points_flat, idx.reshape(B * G * K),
                           B * G * K, 2048).reshape(B, GK, C)
    out = _dense_stages(grouped2d, new_points, affine_alpha, affine_beta,
                        w1, g1, b1, w2, g2, b2)
    return (new_xyz, out)
